# baseline (speedup 1.0000x reference)
import jax
import jax.numpy as jnp
from jax.experimental import pallas as pl
from jax.experimental.pallas import tpu as pltpu

NUM_REL = 20000
NUM_OBJ = 1000
NUM_REL_CLS = 51
NUM_OBJ_CLS = 151


def _rowsum(e):
    rows, c = e.shape
    p = ((c + 7) // 8) * 8
    if p != c:
        e = jnp.concatenate([e, jnp.zeros((rows, p - c), e.dtype)], axis=1)
    acc = e[:, 0:8]
    for k in range(1, p // 8):
        acc = acc + e[:, 8 * k:8 * k + 8]
    s4 = acc[:, 0:4] + acc[:, 4:8]
    s2 = s4[:, 0:2] + s4[:, 2:4]
    return s2[:, 0:1] + s2[:, 1:2]


def _branch_body(x_ref, prob_ref, scores_ref, cls_ref):
    x = x_ref[...]
    m = jnp.max(x, axis=-1, keepdims=True)
    e = jnp.exp(x - m)
    p = e / _rowsum(e)
    prob_ref[...] = p
    cols = jax.lax.broadcasted_iota(jnp.int32, p.shape, 1)
    pm = jnp.where(cols >= 1, p, -1.0)
    sc = jnp.max(pm, axis=-1)
    scores_ref[...] = sc[:, None]
    cls_ref[...] = jnp.min(
        jnp.where(pm == sc[:, None], cols, x.shape[1]), axis=-1)[:, None]


_REL_BLK = 2000


def _dense_stage(rel_logits, obj_logits):
    rel_class_prob, rel_scores, rel_class = pl.pallas_call(
        _branch_body,
        grid=(NUM_REL // _REL_BLK,),
        in_specs=[pl.BlockSpec((_REL_BLK, NUM_REL_CLS), lambda i: (i, 0))],
        out_specs=(
            pl.BlockSpec((_REL_BLK, NUM_REL_CLS), lambda i: (i, 0)),
            pl.BlockSpec((_REL_BLK, 1), lambda i: (i, 0)),
            pl.BlockSpec((_REL_BLK, 1), lambda i: (i, 0)),
        ),
        out_shape=(
            jax.ShapeDtypeStruct((NUM_REL, NUM_REL_CLS), jnp.float32),
            jax.ShapeDtypeStruct((NUM_REL, 1), jnp.float32),
            jax.ShapeDtypeStruct((NUM_REL, 1), jnp.int32),
        ),
    )(rel_logits)
    rel_scores = rel_scores[:, 0]
    rel_class = rel_class[:, 0]
    _, obj_scores, obj_pred = pl.pallas_call(
        _branch_body,
        out_shape=(
            jax.ShapeDtypeStruct((NUM_OBJ, NUM_OBJ_CLS), jnp.float32),
            jax.ShapeDtypeStruct((NUM_OBJ, 1), jnp.float32),
            jax.ShapeDtypeStruct((NUM_OBJ, 1), jnp.int32),
        ),
    )(obj_logits)
    obj_scores = obj_scores[:, 0]
    obj_pred = obj_pred[:, 0]
    return rel_class_prob, rel_scores, rel_class, obj_scores, obj_pred


def kernel(rel_logits, obj_logits, rel_pair_idxs):
    rel_class_prob, rel_scores, rel_class, obj_scores, obj_pred = _dense_stage(
        rel_logits, obj_logits)
    s0 = obj_scores[rel_pair_idxs[:, 0]]
    s1 = obj_scores[rel_pair_idxs[:, 1]]
    triple = rel_scores * s0 * s1
    order = jnp.argsort(-triple)
    return (obj_pred, obj_scores, rel_pair_idxs[order], rel_class_prob[order],
            rel_class[order], triple[order])


# speedup vs baseline: 1.1736x; 1.1736x over previous
import functools

import jax
import jax.numpy as jnp
from jax import lax
from jax.experimental import pallas as pl
from jax.experimental.pallas import tpu as pltpu
from jax.experimental.pallas import tpu_sc as plsc

NUM_REL = 20000
NUM_OBJ = 1000
NUM_REL_CLS = 51
NUM_OBJ_CLS = 151

NP = 20480
NT = 16
CH = NP // NT
LPT = CH // 16
NSUB = CH // 128
PROBW = 64


def _rowsum(e):
    rows, c = e.shape
    p = ((c + 7) // 8) * 8
    if p != c:
        e = jnp.concatenate([e, jnp.zeros((rows, p - c), e.dtype)], axis=1)
    acc = e[:, 0:8]
    for k in range(1, p // 8):
        acc = acc + e[:, 8 * k:8 * k + 8]
    s4 = acc[:, 0:4] + acc[:, 4:8]
    s2 = s4[:, 0:2] + s4[:, 2:4]
    return s2[:, 0:1] + s2[:, 1:2]


def _branch_body(x_ref, prob_ref, scores_ref, cls_ref):
    x = x_ref[...]
    m = jnp.max(x, axis=-1, keepdims=True)
    e = jnp.exp(x - m)
    p = e / _rowsum(e)
    pw = prob_ref.shape[-1]
    if pw != x.shape[1]:
        prob_ref[...] = jnp.concatenate(
            [p, jnp.zeros((x.shape[0], pw - x.shape[1]), p.dtype)], axis=1)
    else:
        prob_ref[...] = p
    cols = jax.lax.broadcasted_iota(jnp.int32, p.shape, 1)
    pm = jnp.where(cols >= 1, p, -1.0)
    sc = jnp.max(pm, axis=-1)
    scores_ref[...] = sc[:, None]
    cls_ref[...] = jnp.min(
        jnp.where(pm == sc[:, None], cols, x.shape[1]), axis=-1)[:, None]


_REL_BLK = 2000


def _dense_stage(rel_logits, obj_logits):
    rel_class_prob, rel_scores, rel_class = pl.pallas_call(
        _branch_body,
        grid=(NUM_REL // _REL_BLK,),
        in_specs=[pl.BlockSpec((_REL_BLK, NUM_REL_CLS), lambda i: (i, 0))],
        out_specs=(
            pl.BlockSpec((_REL_BLK, PROBW), lambda i: (i, 0)),
            pl.BlockSpec((_REL_BLK, 1), lambda i: (i, 0)),
            pl.BlockSpec((_REL_BLK, 1), lambda i: (i, 0)),
        ),
        out_shape=(
            jax.ShapeDtypeStruct((NUM_REL, PROBW), jnp.float32),
            jax.ShapeDtypeStruct((NUM_REL, 1), jnp.float32),
            jax.ShapeDtypeStruct((NUM_REL, 1), jnp.int32),
        ),
    )(rel_logits)
    rel_scores = rel_scores[:, 0]
    rel_class = rel_class[:, 0]
    _, obj_scores, obj_pred = pl.pallas_call(
        _branch_body,
        out_shape=(
            jax.ShapeDtypeStruct((NUM_OBJ, NUM_OBJ_CLS), jnp.float32),
            jax.ShapeDtypeStruct((NUM_OBJ, 1), jnp.float32),
            jax.ShapeDtypeStruct((NUM_OBJ, 1), jnp.int32),
        ),
    )(obj_logits)
    obj_scores = obj_scores[:, 0]
    obj_pred = obj_pred[:, 0]
    return rel_class_prob, rel_scores, rel_class, obj_scores, obj_pred


def _sc_sort_gather(rs_pad, obj_scores, p0_pad, p1_pad, cls_pad, prob):
    mesh = plsc.VectorSubcoreMesh(
        core_axis_name="c", subcore_axis_name="s", num_cores=1)

    @functools.partial(
        pl.kernel,
        out_type=(
            jax.ShapeDtypeStruct((NP,), jnp.float32),
            jax.ShapeDtypeStruct((NP,), jnp.int32),
            jax.ShapeDtypeStruct((NP,), jnp.int32),
            jax.ShapeDtypeStruct((NP,), jnp.int32),
            jax.ShapeDtypeStruct((NP, PROBW), jnp.float32),
        ),
        mesh=mesh,
        compiler_params=pltpu.CompilerParams(
            needs_layout_passes=False, use_tc_tiling_on_sc=False),
        scratch_types=[
            pltpu.VMEM((NUM_OBJ,), jnp.float32),
            pltpu.VMEM((CH,), jnp.float32),
            pltpu.VMEM((CH,), jnp.int32),
            pltpu.VMEM((CH,), jnp.int32),
            pltpu.VMEM((CH,), jnp.int32),
            pltpu.VMEM((CH,), jnp.int32),
            pltpu.VMEM((4096,), jnp.int32),
            pltpu.VMEM((256,), jnp.int32),
            pltpu.VMEM((NT, 256), jnp.int32),
            pltpu.VMEM((4096,), jnp.int32),
            pltpu.VMEM((4096,), jnp.int32),
            pltpu.VMEM((NSUB, 128), jnp.int32),
            pltpu.VMEM((NSUB, 128), jnp.int32),
            pltpu.VMEM((CH,), jnp.float32),
            pltpu.VMEM((CH,), jnp.int32),
            pltpu.VMEM((CH, PROBW), jnp.float32),
            pltpu.VMEM_SHARED((NP,), jnp.int32),
            pltpu.VMEM_SHARED((NP,), jnp.int32),
            pltpu.VMEM_SHARED((NP,), jnp.int32),
            pltpu.VMEM_SHARED((NP,), jnp.int32),
            pltpu.VMEM_SHARED((NT, 256), jnp.int32),
        ],
    )
    def k(rs_hbm, obj_hbm, p0_hbm, p1_hbm, cls_hbm, prob_hbm,
          t_out, p0_out, p1_out, cls_out, prob_out,
          obj_v, rs_v, p0_v, p1_v, key_v, val_v,
          hist_v, tt_v, grid_v, scanrow_v, offs_v,
          pos2_v, idx2_v, f_v, g_v, rows_v,
          ka_sh, kb_sh, va_sh, vb_sh, hist_sh):
        t = lax.axis_index("s")
        base = t * CH
        lane = lax.iota(jnp.int32, 16)
        ones16 = jnp.ones((16,), jnp.int32)
        zeros16 = jnp.zeros((16,), jnp.int32)

        pltpu.sync_copy(obj_hbm, obj_v)
        pltpu.sync_copy(rs_hbm.at[pl.ds(base, CH)], rs_v)
        pltpu.sync_copy(p0_hbm.at[pl.ds(base, CH)], p0_v)
        pltpu.sync_copy(p1_hbm.at[pl.ds(base, CH)], p1_v)

        def ph0(j, carry):
            epos = j * 16 + lane
            i0 = plsc.load_gather(p0_v, [epos])
            i1 = plsc.load_gather(p1_v, [epos])
            s0 = plsc.load_gather(obj_v, [i0])
            s1 = plsc.load_gather(obj_v, [i1])
            rs = plsc.load_gather(rs_v, [epos])
            tr = (rs * s0) * s1
            bits = plsc.bitcast(tr, jnp.int32)
            gidx = base + epos
            pad = gidx >= NUM_REL
            key = jnp.where(pad, jnp.int32(0x7FFFFFFF),
                            jnp.int32(0x7FFFFFFF) - bits)
            plsc.store_scatter(key_v, [epos], key)
            plsc.store_scatter(val_v, [epos], jnp.where(pad, zeros16, gidx))
            return carry
        lax.fori_loop(0, LPT * 16 // 16, ph0, jnp.int32(0))

        bufs = [(ka_sh, va_sh), (kb_sh, vb_sh)]
        for p in range(4):
            dst = bufs[p % 2]
            if p > 0:
                src = bufs[(p - 1) % 2]
                pltpu.sync_copy(src[0].at[pl.ds(base, CH)], key_v)
                pltpu.sync_copy(src[1].at[pl.ds(base, CH)], val_v)
            shift = 8 * p

            def clr(i, carry):
                plsc.store_scatter(hist_v, [i * 16 + lane], zeros16)
                return carry
            lax.fori_loop(0, 256, clr, jnp.int32(0))

            def hst(j, carry, shift=shift):
                epos = lane * LPT + j
                kk = plsc.load_gather(key_v, [epos])
                d = jnp.bitwise_and(lax.shift_right_logical(kk, shift), 255)
                plsc.addupdate_scatter(hist_v, [d * 16 + lane], ones16)
                return carry
            lax.fori_loop(0, LPT, hst, jnp.int32(0))

            def tot(g, carry):
                dvec = g * 16 + lane
                acc = zeros16
                for l in range(16):
                    acc = acc + plsc.load_gather(hist_v, [dvec * 16 + l])
                plsc.store_scatter(tt_v, [dvec], acc)
                return carry
            lax.fori_loop(0, 16, tot, jnp.int32(0))

            pltpu.sync_copy(tt_v, hist_sh.at[t])
            plsc.subcore_barrier()
            pltpu.sync_copy(hist_sh, grid_v)

            def scn(d, running):
                v = plsc.load_gather(grid_v, [lane, zeros16 + d])
                cs = plsc.cumsum(v)
                basev = (cs - v) + running
                plsc.store_scatter(scanrow_v, [d * 16 + lane], basev)
                lane_h = plsc.load_gather(hist_v, [d * 16 + lane])
                lane_cs = plsc.cumsum(lane_h)
                own = plsc.load_gather(scanrow_v, [zeros16 + (d * 16 + t)])
                plsc.store_scatter(offs_v, [d * 16 + lane],
                                   own + (lane_cs - lane_h))
                return running + jnp.sum(v)
            lax.fori_loop(0, 256, scn, jnp.int32(0))

            def prm(j, carry, shift=shift):
                epos = lane * LPT + j
                kk = plsc.load_gather(key_v, [epos])
                d = jnp.bitwise_and(lax.shift_right_logical(kk, shift), 255)
                oidx = d * 16 + lane
                pos = plsc.load_gather(offs_v, [oidx])
                plsc.store_scatter(offs_v, [oidx], pos + 1)
                plsc.store_scatter(
                    pos2_v,
                    [lax.shift_right_logical(epos, 7),
                     jnp.bitwise_and(epos, 127)], pos)
                return carry
            lax.fori_loop(0, LPT, prm, jnp.int32(0))

            for i in range(NSUB):
                pltpu.sync_copy(key_v.at[pl.ds(i * 128, 128)],
                                dst[0].at[pos2_v.at[i]])
                pltpu.sync_copy(val_v.at[pl.ds(i * 128, 128)],
                                dst[1].at[pos2_v.at[i]])
            plsc.subcore_barrier()

        fk, fv = bufs[3 % 2]
        pltpu.sync_copy(fk.at[pl.ds(base, CH)], key_v)
        pltpu.sync_copy(fv.at[pl.ds(base, CH)], val_v)

        def fin(j, carry):
            epos = j * 16 + lane
            kk = plsc.load_gather(key_v, [epos])
            tr = plsc.bitcast(jnp.int32(0x7FFFFFFF) - kk, jnp.float32)
            plsc.store_scatter(f_v, [epos], tr)
            vv = plsc.load_gather(val_v, [epos])
            plsc.store_scatter(
                idx2_v,
                [lax.shift_right_logical(epos, 7),
                 jnp.bitwise_and(epos, 127)], vv)
            return carry
        lax.fori_loop(0, CH // 16, fin, jnp.int32(0))
        pltpu.sync_copy(f_v, t_out.at[pl.ds(base, CH)])

        for src_hbm, out_hbm in ((p0_hbm, p0_out), (p1_hbm, p1_out),
                                 (cls_hbm, cls_out)):
            for i in range(NSUB):
                pltpu.sync_copy(src_hbm.at[idx2_v.at[i]],
                                g_v.at[pl.ds(i * 128, 128)])
            pltpu.sync_copy(g_v, out_hbm.at[pl.ds(base, CH)])
        for i in range(NSUB):
            pltpu.sync_copy(prob_hbm.at[idx2_v.at[i]],
                            rows_v.at[pl.ds(i * 128, 128)])
        pltpu.sync_copy(rows_v, prob_out.at[pl.ds(base, CH)])

    return k(rs_pad, obj_scores, p0_pad, p1_pad, cls_pad, prob)


def kernel(rel_logits, obj_logits, rel_pair_idxs):
    rel_class_prob, rel_scores, rel_class, obj_scores, obj_pred = _dense_stage(
        rel_logits, obj_logits)
    pad = NP - NUM_REL
    zi = jnp.zeros((pad,), jnp.int32)
    rs_pad = jnp.concatenate([rel_scores, jnp.zeros((pad,), jnp.float32)])
    p0_pad = jnp.concatenate([rel_pair_idxs[:, 0], zi])
    p1_pad = jnp.concatenate([rel_pair_idxs[:, 1], zi])
    cls_pad = jnp.concatenate([rel_class, zi])
    ts, p0s, p1s, clss, probs = _sc_sort_gather(
        rs_pad, obj_scores, p0_pad, p1_pad, cls_pad, rel_class_prob)
    pair_sorted = jnp.stack([p0s[:NUM_REL], p1s[:NUM_REL]], axis=1)
    return (obj_pred, obj_scores, pair_sorted,
            probs[:NUM_REL, :NUM_REL_CLS], clss[:NUM_REL], ts[:NUM_REL])
